# baseline (device time: 28474 ns/iter reference)
import jax
import jax.numpy as jnp
from jax import lax
from jax.experimental import pallas as pl
from jax.experimental.pallas import tpu as pltpu

N_DEV = 4


def kernel(table, idx):
    v_per, d = table.shape
    n = idx.shape[0]
    h = n // 2
    vh = v_per // 2
    idx2 = idx.reshape(n, 1)

    def body(
        table_ref, idx_ref, out_ref,
        tvmem, acc_a, acc_b, rbuf, load_sems, send_sems, recv_sems,
    ):
        my = lax.axis_index("i")
        p_a = my ^ 1
        p_b = 3 - my

        loads = []
        for k in range(2):
            cp = pltpu.make_async_copy(
                table_ref.at[pl.ds(k * vh, vh), :],
                tvmem.at[k],
                load_sems.at[k],
            )
            cp.start()
            loads.append(cp)

        barrier_sem = pltpu.get_barrier_semaphore()
        for nbr in [p_a, p_b]:
            pl.semaphore_signal(
                barrier_sem, inc=1,
                device_id=(nbr,), device_id_type=pl.DeviceIdType.MESH,
            )
        pl.semaphore_wait(barrier_sem, 2)

        def exchange(src, ph, half, tgt):
            return pltpu.make_async_remote_copy(
                src_ref=src,
                dst_ref=rbuf.at[ph, half],
                send_sem=send_sems.at[ph, half],
                recv_sem=recv_sems.at[ph, half],
                device_id=(tgt,),
                device_id_type=pl.DeviceIdType.MESH,
            )

        local = idx_ref[...] - my * v_per
        iota = lax.broadcasted_iota(jnp.int32, (h, v_per), 1)

        loads[0].wait()
        t0 = tvmem[0].astype(jnp.bfloat16)
        pa0 = jnp.dot(
            (iota[:, :vh] == local[:h]).astype(jnp.bfloat16),
            t0, preferred_element_type=jnp.float32,
        )
        loads[1].wait()
        t1 = tvmem[1].astype(jnp.bfloat16)
        pa1 = jnp.dot(
            (iota[:, vh:] == local[:h] - vh).astype(jnp.bfloat16),
            t1, preferred_element_type=jnp.float32,
        )
        acc_a[...] = (pa0 + pa1).astype(jnp.bfloat16)
        a0 = exchange(acc_a, 0, 0, p_a)
        a0.start()

        pb0 = jnp.dot(
            (iota[:, :vh] == local[h:]).astype(jnp.bfloat16),
            t0, preferred_element_type=jnp.float32,
        )
        pb1 = jnp.dot(
            (iota[:, vh:] == local[h:] - vh).astype(jnp.bfloat16),
            t1, preferred_element_type=jnp.float32,
        )
        acc_b[...] = (pb0 + pb1).astype(jnp.bfloat16)
        b0 = exchange(acc_b, 0, 1, p_b)
        b0.start()

        a0.wait()
        acc_a[...] += rbuf[0, 0]
        a1 = exchange(acc_a, 1, 0, p_b)
        a1.start()

        b0.wait()
        acc_b[...] += rbuf[0, 1]
        b1 = exchange(acc_b, 1, 1, p_a)
        b1.start()

        a1.wait()
        out_ref[:h, :] = acc_a[...] + rbuf[1, 0]
        b1.wait()
        out_ref[h:, :] = acc_b[...] + rbuf[1, 1]

    return pl.pallas_call(
        body,
        out_shape=jax.ShapeDtypeStruct((n, d), jnp.bfloat16),
        in_specs=[
            pl.BlockSpec(memory_space=pl.ANY),
            pl.BlockSpec(memory_space=pltpu.VMEM),
        ],
        out_specs=pl.BlockSpec(memory_space=pltpu.VMEM),
        scratch_shapes=[
            pltpu.VMEM((2, vh, d), jnp.float32),
            pltpu.VMEM((h, d), jnp.bfloat16),
            pltpu.VMEM((h, d), jnp.bfloat16),
            pltpu.VMEM((2, 2, h, d), jnp.bfloat16),
            pltpu.SemaphoreType.DMA((2,)),
            pltpu.SemaphoreType.DMA((2, 2)),
            pltpu.SemaphoreType.DMA((2, 2)),
        ],
        compiler_params=pltpu.CompilerParams(collective_id=0),
    )(table, idx2)


# device time: 28390 ns/iter; 1.0030x vs baseline; 1.0030x over previous
import jax
import jax.numpy as jnp
from jax import lax
from jax.experimental import pallas as pl
from jax.experimental.pallas import tpu as pltpu

N_DEV = 4


def kernel(table, idx):
    v_per, d = table.shape
    n = idx.shape[0]
    h = n // 2
    vh = v_per // 2
    idx2 = idx.reshape(n, 1)

    def body(
        table_ref, idx_ref, out_ref,
        tvmem, acc_a, acc_b, rbuf, load_sems, send_sems, recv_sems,
    ):
        my = lax.axis_index("i")
        p_a = my ^ 1
        p_b = 3 - my

        loads = []
        for k in range(2):
            cp = pltpu.make_async_copy(
                table_ref.at[pl.ds(k * vh, vh), :],
                tvmem.at[k],
                load_sems.at[k],
            )
            cp.start()
            loads.append(cp)

        barrier_sem = pltpu.get_barrier_semaphore()
        for nbr in [p_a, p_b]:
            pl.semaphore_signal(
                barrier_sem, inc=1,
                device_id=(nbr,), device_id_type=pl.DeviceIdType.MESH,
            )
        pl.semaphore_wait(barrier_sem, 2)

        def exchange(src, ph, half, tgt):
            return pltpu.make_async_remote_copy(
                src_ref=src,
                dst_ref=rbuf.at[ph, half],
                send_sem=send_sems.at[ph, half],
                recv_sem=recv_sems.at[ph, half],
                device_id=(tgt,),
                device_id_type=pl.DeviceIdType.MESH,
            )

        local = idx_ref[...] - my * v_per
        iota = lax.broadcasted_iota(jnp.int32, (h, v_per), 1)

        loads[0].wait()
        t0 = tvmem[0].astype(jnp.bfloat16)
        pa0 = jnp.dot(
            (iota[:, :vh] == local[:h]).astype(jnp.bfloat16),
            t0, preferred_element_type=jnp.float32,
        )
        loads[1].wait()
        t1 = tvmem[1].astype(jnp.bfloat16)
        pa1 = jnp.dot(
            (iota[:, vh:] == local[:h]).astype(jnp.bfloat16),
            t1, preferred_element_type=jnp.float32,
        )
        acc_a[...] = (pa0 + pa1).astype(jnp.bfloat16)
        a0 = exchange(acc_a, 0, 0, p_a)
        a0.start()

        pb0 = jnp.dot(
            (iota[:, :vh] == local[h:]).astype(jnp.bfloat16),
            t0, preferred_element_type=jnp.float32,
        )
        pb1 = jnp.dot(
            (iota[:, vh:] == local[h:]).astype(jnp.bfloat16),
            t1, preferred_element_type=jnp.float32,
        )
        acc_b[...] = (pb0 + pb1).astype(jnp.bfloat16)
        b0 = exchange(acc_b, 0, 1, p_b)
        b0.start()

        a0.wait()
        acc_a[...] += rbuf[0, 0]
        a1 = exchange(acc_a, 1, 0, p_b)
        a1.start()

        b0.wait()
        acc_b[...] += rbuf[0, 1]
        b1 = exchange(acc_b, 1, 1, p_a)
        b1.start()

        a1.wait()
        out_ref[:h, :] = acc_a[...] + rbuf[1, 0]
        b1.wait()
        out_ref[h:, :] = acc_b[...] + rbuf[1, 1]

    return pl.pallas_call(
        body,
        out_shape=jax.ShapeDtypeStruct((n, d), jnp.bfloat16),
        in_specs=[
            pl.BlockSpec(memory_space=pl.ANY),
            pl.BlockSpec(memory_space=pltpu.VMEM),
        ],
        out_specs=pl.BlockSpec(memory_space=pltpu.VMEM),
        scratch_shapes=[
            pltpu.VMEM((2, vh, d), jnp.float32),
            pltpu.VMEM((h, d), jnp.bfloat16),
            pltpu.VMEM((h, d), jnp.bfloat16),
            pltpu.VMEM((2, 2, h, d), jnp.bfloat16),
            pltpu.SemaphoreType.DMA((2,)),
            pltpu.SemaphoreType.DMA((2, 2)),
            pltpu.SemaphoreType.DMA((2, 2)),
        ],
        compiler_params=pltpu.CompilerParams(collective_id=0),
    )(table, idx2)
